# baseline (speedup 1.0000x reference)
import jax
import jax.numpy as jnp
from jax import lax
from jax.experimental import pallas as pl
from jax.experimental.pallas import tpu as pltpu

_MARGIN = 0.2
_N = 128
_SLAB = 8


def _body(x_ref, lab_row_ref, lab_col_ref, out_ref):
    x = x_ref[...]
    lab_row = lab_row_ref[...]
    lab_col = lab_col_ref[...]

    g = lax.dot_general(x, x, (((1,), (1,)), ((), ())),
                        preferred_element_type=jnp.float32)
    xsq = x * x
    nrm_col = jnp.sum(xsq, axis=1, keepdims=True)
    ones_row = jnp.ones((1, _N), dtype=jnp.float32)
    nrm_row = lax.dot_general(ones_row, xsq, (((1,), (1,)), ((), ())),
                              preferred_element_type=jnp.float32)
    d = nrm_col + nrm_row - 2.0 * g

    same = lab_col == lab_row
    rowi = lax.broadcasted_iota(jnp.int32, (_N, _N), 0)
    colj = lax.broadcasted_iota(jnp.int32, (_N, _N), 1)
    pairf = jnp.where(same & (rowi < colj), 1.0, 0.0)
    negf = jnp.where(same, 0.0, 1.0)
    w_col = jnp.sum(jnp.where(same, 1.0, 0.0), axis=1, keepdims=True)
    pmw = pairf * w_col

    p_col = jnp.sum(pairf, axis=1, keepdims=True)
    n_col = jnp.sum(negf, axis=1, keepdims=True)
    cnt = jnp.sum(w_col * p_col * n_col)

    dm = d + _MARGIN
    tot = jnp.float32(0.0)
    for s in range(_N // _SLAB):
        sl = slice(s * _SLAB, (s + 1) * _SLAB)
        da_p = dm[sl, :]
        da_n = d[sl, :]
        z = da_p[:, :, None] - da_n[:, None, :]
        r = jnp.maximum(z, 0.0) * negf[sl, :][:, None, :]
        rn = jnp.sum(r, axis=2)
        tot = tot + jnp.sum(rn * pmw[sl, :])

    out_ref[0, 0] = jnp.where(cnt > 0.0, tot / cnt, 0.0)


def kernel(x, labels):
    labf = labels.astype(jnp.float32)
    out = pl.pallas_call(
        _body,
        out_shape=jax.ShapeDtypeStruct((1, 1), jnp.float32),
        in_specs=[
            pl.BlockSpec(memory_space=pltpu.VMEM),
            pl.BlockSpec(memory_space=pltpu.VMEM),
            pl.BlockSpec(memory_space=pltpu.VMEM),
        ],
        out_specs=pl.BlockSpec(memory_space=pltpu.SMEM),
    )(x, labf.reshape(1, _N), labf.reshape(_N, 1))
    return out.reshape(())


# speedup vs baseline: 1.0033x; 1.0033x over previous
import jax
import jax.numpy as jnp
from jax import lax
from jax.experimental import pallas as pl
from jax.experimental.pallas import tpu as pltpu

_MARGIN = 0.2
_N = 128
_SLAB = 8


def _body(x_ref, lab_row_ref, lab_col_ref, out_ref):
    x = x_ref[...]
    lab_row = lab_row_ref[...]
    lab_col = lab_col_ref[...]

    g = lax.dot_general(x, x, (((1,), (1,)), ((), ())),
                        preferred_element_type=jnp.float32)
    xsq = x * x
    nrm_col = jnp.sum(xsq, axis=1, keepdims=True)
    ones_row = jnp.ones((1, _N), dtype=jnp.float32)
    nrm_row = lax.dot_general(ones_row, xsq, (((1,), (1,)), ((), ())),
                              preferred_element_type=jnp.float32)
    d = nrm_col + nrm_row - 2.0 * g

    same = lab_col == lab_row
    rowi = lax.broadcasted_iota(jnp.int32, (_N, _N), 0)
    colj = lax.broadcasted_iota(jnp.int32, (_N, _N), 1)
    pairf = jnp.where(same & (rowi < colj), 1.0, 0.0)
    negf = jnp.where(same, 0.0, 1.0)
    w_col = jnp.sum(jnp.where(same, 1.0, 0.0), axis=1, keepdims=True)
    pmw = pairf * w_col

    p_col = jnp.sum(pairf, axis=1, keepdims=True)
    n_col = jnp.sum(negf, axis=1, keepdims=True)
    cnt = jnp.sum(w_col * p_col * n_col)

    dm = d + _MARGIN
    dbig = d + jnp.where(same, 1e9, 0.0)
    tot = jnp.float32(0.0)
    for s in range(_N // _SLAB):
        sl = slice(s * _SLAB, (s + 1) * _SLAB)
        z = dm[sl, :][:, :, None] - dbig[sl, :][:, None, :]
        rn = jnp.sum(jnp.maximum(z, 0.0), axis=2)
        tot = tot + jnp.sum(rn * pmw[sl, :])

    out_ref[0, 0] = jnp.where(cnt > 0.0, tot / cnt, 0.0)


def kernel(x, labels):
    labf = labels.astype(jnp.float32)
    out = pl.pallas_call(
        _body,
        out_shape=jax.ShapeDtypeStruct((1, 1), jnp.float32),
        in_specs=[
            pl.BlockSpec(memory_space=pltpu.VMEM),
            pl.BlockSpec(memory_space=pltpu.VMEM),
            pl.BlockSpec(memory_space=pltpu.VMEM),
        ],
        out_specs=pl.BlockSpec(memory_space=pltpu.SMEM),
    )(x, labf.reshape(1, _N), labf.reshape(_N, 1))
    return out.reshape(())


# speedup vs baseline: 2.2165x; 2.2093x over previous
import jax
import jax.numpy as jnp
from jax import lax
from jax.experimental import pallas as pl
from jax.experimental.pallas import tpu as pltpu

_MARGIN = 0.2
_N = 128
_SLAB = 8


def _body(x_ref, lab_row_ref, lab_col_ref, out_ref):
    x = x_ref[...]
    lab_row = lab_row_ref[...]
    lab_col = lab_col_ref[...]

    g = lax.dot_general(x, x, (((1,), (1,)), ((), ())),
                        preferred_element_type=jnp.float32)
    xsq = x * x
    nrm_col = jnp.sum(xsq, axis=1, keepdims=True)
    ones_row = jnp.ones((1, _N), dtype=jnp.float32)
    nrm_row = lax.dot_general(ones_row, xsq, (((1,), (1,)), ((), ())),
                              preferred_element_type=jnp.float32)
    d = nrm_col + nrm_row - 2.0 * g

    same = lab_col == lab_row
    rowi = lax.broadcasted_iota(jnp.int32, (_N, _N), 0)
    colj = lax.broadcasted_iota(jnp.int32, (_N, _N), 1)
    pairf = jnp.where(same & (rowi < colj), 1.0, 0.0)
    negf = jnp.where(same, 0.0, 1.0)
    w_col = jnp.sum(jnp.where(same, 1.0, 0.0), axis=1, keepdims=True)

    p_col = jnp.sum(pairf, axis=1, keepdims=True)
    n_col = jnp.sum(negf, axis=1, keepdims=True)
    cnt = jnp.sum(w_col * p_col * n_col)

    big = jnp.float32(1e9)
    w_row = lax.dot_general(ones_row, jnp.where(same, 1.0, 0.0),
                            (((1,), (1,)), ((), ())),
                            preferred_element_type=jnp.float32)
    dmw = (d + _MARGIN - jnp.where(same & (rowi < colj), 0.0, big)) * w_col
    dbigw = (d + jnp.where(same, big, 0.0)) * w_row
    acc = jnp.zeros((_N, _N), jnp.float32)
    for a in range(_N):
        z = dmw[a:a + 1, :] - dbigw[:, a:a + 1]
        acc = acc + jnp.maximum(z, 0.0)
    tot = jnp.sum(acc)

    out_ref[0, 0] = jnp.where(cnt > 0.0, tot / cnt, 0.0)


def kernel(x, labels):
    labf = labels.astype(jnp.float32)
    out = pl.pallas_call(
        _body,
        out_shape=jax.ShapeDtypeStruct((1, 1), jnp.float32),
        in_specs=[
            pl.BlockSpec(memory_space=pltpu.VMEM),
            pl.BlockSpec(memory_space=pltpu.VMEM),
            pl.BlockSpec(memory_space=pltpu.VMEM),
        ],
        out_specs=pl.BlockSpec(memory_space=pltpu.SMEM),
    )(x, labf.reshape(1, _N), labf.reshape(_N, 1))
    return out.reshape(())
